# baseline (speedup 1.0000x reference)
import jax
import jax.numpy as jnp
from jax import lax
from jax.experimental import pallas as pl
from jax.experimental.pallas import tpu as pltpu
from jax.experimental.pallas import tpu_sc as plsc

N_TOKENS = 1024
DIM = 256
N_Q = 4
BINS = 512

_GW = 128


def _level_body(residual, cb):
    ones8 = jnp.ones((8, DIM), jnp.float32)
    chalf8 = 0.5 * lax.dot_general(
        ones8, cb * cb,
        dimension_numbers=(((1,), (1,)), ((), ())),
        preferred_element_type=jnp.float32,
        precision=lax.Precision.HIGHEST,
    )
    dots = lax.dot_general(
        residual, cb,
        dimension_numbers=(((1,), (1,)), ((), ())),
        preferred_element_type=jnp.float32,
        precision=lax.Precision.HIGHEST,
    )
    scores = dots - chalf8[0:1, :]
    maxs = jnp.max(scores, axis=1, keepdims=True)
    iota = lax.broadcasted_iota(jnp.int32, scores.shape, 1)
    return jnp.min(jnp.where(scores == maxs, iota, BINS),
                   axis=1, keepdims=True)


def _tc_first(h_ref, cb_ref, idx_ref):
    idx_ref[:] = _level_body(h_ref[:], cb_ref[:])


def _tc_mid(r_ref, cprev_ref, cb_ref, idx_ref, rnew_ref):
    residual = r_ref[:] - cprev_ref[:]
    rnew_ref[:] = residual
    idx_ref[:] = _level_body(residual, cb_ref[:])


def _tc_final(h_ref, r_ref, cprev_ref, quant_ref):
    quant_ref[:] = h_ref[:] - (r_ref[:] - cprev_ref[:])


_IDX_TYPE = jax.ShapeDtypeStruct((N_TOKENS, 1), jnp.int32)
_RES_TYPE = jax.ShapeDtypeStruct((N_TOKENS, DIM), jnp.float32)


def _sc_gather(cb_level, idx_col):
    idx_row = idx_col.reshape(1, N_TOKENS)

    @pl.kernel(out_type=_RES_TYPE,
               mesh=plsc.VectorSubcoreMesh(core_axis_name="core",
                                           subcore_axis_name="subcore"))
    def k(cb_hbm, i_hbm, o_hbm):
        def body(i_vmem, o_vmem):
            pltpu.sync_copy(cb_hbm.at[i_vmem.at[0]], o_vmem)

        pltpu.emit_pipeline(
            body,
            grid=(N_TOKENS // _GW,),
            in_specs=[pl.BlockSpec((1, _GW), index_map=lambda i: (0, i))],
            out_specs=[pl.BlockSpec((_GW, DIM), index_map=lambda i: (i, 0))],
            core_axis_name="subcore",
            dimension_semantics=(pltpu.PARALLEL,),
        )(i_hbm, o_hbm)

    return k(cb_level, idx_row)


def kernel(hidden_states, codebooks):
    h = hidden_states
    idx0 = pl.pallas_call(_tc_first, out_shape=_IDX_TYPE)(h, codebooks[0])
    chosen = _sc_gather(codebooks[0], idx0)
    residual = h
    idx_cols = [idx0]
    for i in range(1, N_Q):
        idx_i, residual = pl.pallas_call(
            _tc_mid, out_shape=[_IDX_TYPE, _RES_TYPE],
        )(residual, chosen, codebooks[i])
        chosen = _sc_gather(codebooks[i], idx_i)
        idx_cols.append(idx_i)
    quant = pl.pallas_call(_tc_final, out_shape=_RES_TYPE)(h, residual, chosen)
    codes = jnp.concatenate(idx_cols, axis=1).T
    return codes, quant


# speedup vs baseline: 4.3956x; 4.3956x over previous
import jax
import jax.numpy as jnp
from jax import lax
from jax.experimental import pallas as pl

N_TOKENS = 1024
DIM = 256
N_Q = 4
BINS = 512
HALF = N_TOKENS // 2


def _split3_bf16(x):
    parts = []
    r = x
    for _ in range(3):
        c = r.astype(jnp.bfloat16)
        parts.append(c)
        r = r - c.astype(jnp.float32)
    return parts


def _dots(residual, cb):
    return lax.dot_general(
        residual, cb,
        dimension_numbers=(((1,), (1,)), ((), ())),
        preferred_element_type=jnp.float32,
        precision=lax.Precision.HIGHEST,
    )


def _select(dots, chalf_row):
    scores = dots - chalf_row
    maxs = jnp.max(scores, axis=1, keepdims=True)
    iota = lax.broadcasted_iota(jnp.int32, scores.shape, 1)
    idx2d = jnp.min(jnp.where(scores == maxs, iota, BINS),
                    axis=1, keepdims=True)
    onehot = (iota == idx2d).astype(jnp.bfloat16)
    return idx2d, onehot


def _gather(onehot, splits, lo):
    chosen = None
    for part in splits:
        g = lax.dot_general(
            onehot, part[lo:lo + BINS],
            dimension_numbers=(((1,), (0,)), ((), ())),
            preferred_element_type=jnp.float32,
        )
        chosen = g if chosen is None else chosen + g
    return chosen


def _rvq_kernel(h_ref, cb_ref, codes_ref, quant_ref):
    ones8 = jnp.ones((8, DIM), jnp.float32)
    cb_all = cb_ref[:].reshape(N_Q * BINS, DIM)
    chalf8 = 0.5 * lax.dot_general(
        ones8, cb_all * cb_all,
        dimension_numbers=(((1,), (1,)), ((), ())),
        preferred_element_type=jnp.float32,
        precision=lax.Precision.HIGHEST,
    )
    splits = _split3_bf16(cb_all)

    res = [h_ref[0:HALF, :], h_ref[HALF:N_TOKENS, :]]
    idx_cols = [[], []]
    for i in range(N_Q):
        cb = cb_ref[i]
        chalf_row = chalf8[0:1, i * BINS:(i + 1) * BINS]
        d0 = _dots(res[0], cb)
        d1 = _dots(res[1], cb)
        idx0, oh0 = _select(d0, chalf_row)
        ch0 = _gather(oh0, splits, i * BINS)
        idx1, oh1 = _select(d1, chalf_row)
        ch1 = _gather(oh1, splits, i * BINS)
        res[0] = res[0] - ch0
        res[1] = res[1] - ch1
        idx_cols[0].append(idx0)
        idx_cols[1].append(idx1)
    codes_ref[0:HALF, :] = jnp.concatenate(idx_cols[0], axis=1)
    codes_ref[HALF:N_TOKENS, :] = jnp.concatenate(idx_cols[1], axis=1)
    quant_ref[0:HALF, :] = h_ref[0:HALF, :] - res[0]
    quant_ref[HALF:N_TOKENS, :] = h_ref[HALF:N_TOKENS, :] - res[1]


def kernel(hidden_states, codebooks):
    codes_t, quant = pl.pallas_call(
        _rvq_kernel,
        out_shape=[
            jax.ShapeDtypeStruct((N_TOKENS, N_Q), jnp.int32),
            jax.ShapeDtypeStruct((N_TOKENS, DIM), jnp.float32),
        ],
    )(hidden_states, codebooks)
    return jnp.transpose(codes_t), quant


# speedup vs baseline: 4.7163x; 1.0729x over previous
import jax
import jax.numpy as jnp
from jax import lax
from jax.experimental import pallas as pl

N_TOKENS = 1024
DIM = 256
N_Q = 4
BINS = 512
NSPLIT = 4
CHUNK = N_TOKENS // NSPLIT


def _split3_bf16(x):
    parts = []
    r = x
    for _ in range(3):
        c = r.astype(jnp.bfloat16)
        parts.append(c)
        r = r - c.astype(jnp.float32)
    return parts


def _dots(residual, cb):
    return lax.dot_general(
        residual, cb,
        dimension_numbers=(((1,), (1,)), ((), ())),
        preferred_element_type=jnp.float32,
        precision=lax.Precision.HIGHEST,
    )


def _select(dots, chalf_row):
    scores = dots - chalf_row
    maxs = jnp.max(scores, axis=1, keepdims=True)
    iota = lax.broadcasted_iota(jnp.int32, scores.shape, 1)
    idx2d = jnp.min(jnp.where(scores == maxs, iota, BINS),
                    axis=1, keepdims=True)
    onehot = (iota == idx2d).astype(jnp.bfloat16)
    return idx2d, onehot


def _gather(onehot, splits, lo):
    chosen = None
    for part in splits:
        g = lax.dot_general(
            onehot, part[lo:lo + BINS],
            dimension_numbers=(((1,), (0,)), ((), ())),
            preferred_element_type=jnp.float32,
        )
        chosen = g if chosen is None else chosen + g
    return chosen


def _rvq_kernel(h_ref, cb_ref, codes_ref, quant_ref):
    ones8 = jnp.ones((8, DIM), jnp.float32)
    cb_all = cb_ref[:].reshape(N_Q * BINS, DIM)
    chalf8 = 0.5 * lax.dot_general(
        ones8, cb_all * cb_all,
        dimension_numbers=(((1,), (1,)), ((), ())),
        preferred_element_type=jnp.float32,
        precision=lax.Precision.HIGHEST,
    )
    splits = _split3_bf16(cb_all)

    res = [h_ref[k * CHUNK:(k + 1) * CHUNK, :] for k in range(NSPLIT)]
    idx_cols = [[] for _ in range(NSPLIT)]
    for i in range(N_Q):
        cb = cb_ref[i]
        chalf_row = chalf8[0:1, i * BINS:(i + 1) * BINS]
        d = [_dots(res[k], cb) for k in range(NSPLIT)]
        for k in range(NSPLIT):
            idx_k, oh_k = _select(d[k], chalf_row)
            ch_k = _gather(oh_k, splits, i * BINS)
            res[k] = res[k] - ch_k
            idx_cols[k].append(idx_k)
    for k in range(NSPLIT):
        lo = k * CHUNK
        codes_ref[lo:lo + CHUNK, :] = jnp.concatenate(idx_cols[k], axis=1)
        quant_ref[lo:lo + CHUNK, :] = h_ref[lo:lo + CHUNK, :] - res[k]


def kernel(hidden_states, codebooks):
    codes_t, quant = pl.pallas_call(
        _rvq_kernel,
        out_shape=[
            jax.ShapeDtypeStruct((N_TOKENS, N_Q), jnp.int32),
            jax.ShapeDtypeStruct((N_TOKENS, DIM), jnp.float32),
        ],
    )(hidden_states, codebooks)
    return jnp.transpose(codes_t), quant
